# baseline (speedup 1.0000x reference)
import functools

import jax
import jax.numpy as jnp
from jax import lax
from jax.experimental import pallas as pl
from jax.experimental.pallas import tpu as pltpu
from jax.experimental.pallas import tpu_sc as plsc

N = 10000
NPAD = 10112
E = 320000
CH = 128
CPT = E // CH
NC, NS = 2, 16
NW = NC * NS
KA, KB = 76, 80
NBUF = 4
STRIPE = NPAD // NS
BN_EPS = 1e-5
BLK = 1000


def _tile_chunks(cid, sid):
    wid = sid * NC + cid
    nch = jnp.where(wid < 15, KA, KB)
    base = jnp.where(wid < 15, wid * KA, 15 * KA + (wid - 15) * KB)
    return nch, base


def _sc_deg_body(edge_hbm, ones_hbm, zeros_hbm, out_hbm, dst_v, ones_v, acc):
    cid = lax.axis_index("c")
    sid = lax.axis_index("s")
    nch, base = _tile_chunks(cid, sid)
    pltpu.sync_copy(edge_hbm.at[1, pl.ds(base, KB)], dst_v)
    pltpu.sync_copy(ones_hbm, ones_v)
    pltpu.sync_copy(zeros_hbm, acc.at[pl.ds(sid * STRIPE, STRIPE)])
    plsc.subcore_barrier()

    def body(c, carry):
        pltpu.sync_copy(ones_v, acc.at[dst_v.at[c]], add=True)
        return carry

    lax.fori_loop(0, nch, body, 0)
    plsc.subcore_barrier()
    pltpu.sync_copy(acc.at[pl.ds(sid * STRIPE, STRIPE)],
                    out_hbm.at[cid, pl.ds(sid * STRIPE, STRIPE)])


def _sc_agg_body(nh, *refs):
    g_hbms = refs[:nh]
    (edge_hbm, zeros_hbm, out_hbm, src_v, dst_v) = refs[nh:nh + 5]
    bufs = refs[nh + 5:nh + 5 + 2 * NBUF]
    acc = refs[nh + 5 + 2 * NBUF]
    semg = refs[nh + 6 + 2 * NBUF:nh + 6 + 4 * NBUF]
    sems = refs[nh + 6 + 4 * NBUF:nh + 6 + 6 * NBUF]
    cid = lax.axis_index("c")
    sid = lax.axis_index("s")
    nch, base = _tile_chunks(cid, sid)
    pltpu.sync_copy(edge_hbm.at[0, pl.ds(base, KB)], src_v)
    pltpu.sync_copy(edge_hbm.at[1, pl.ds(base, KB)], dst_v)
    nb2 = 2 * NBUF

    for h, g_hbm in enumerate(g_hbms):
        pltpu.sync_copy(zeros_hbm, acc.at[pl.ds(sid * STRIPE, STRIPE)])
        plsc.subcore_barrier()
        for b in range(NBUF):
            pltpu.async_copy(g_hbm.at[src_v.at[b]], bufs[b], semg[b])

        def slot(c, j):
            jn = (j + NBUF) % nb2

            @pl.when(c >= NBUF)
            def _():
                pltpu.make_async_copy(bufs[jn], acc.at[dst_v.at[c]],
                                      sems[jn]).wait()

            @pl.when(c + NBUF < nch)
            def _():
                pltpu.async_copy(g_hbm.at[src_v.at[c + NBUF]],
                                 bufs[jn], semg[jn])
            pltpu.make_async_copy(g_hbm.at[src_v.at[c]],
                                  bufs[j], semg[j]).wait()

            @pl.when(c < nch - NBUF)
            def _():
                pltpu.async_copy(bufs[j], acc.at[dst_v.at[c]], sems[j],
                                 add=True)

            @pl.when(c >= nch - NBUF)
            def _():
                pltpu.sync_copy(bufs[j], acc.at[dst_v.at[c]], add=True)

        def body(t, carry):
            for j in range(nb2):
                slot(nb2 * t + j, j)
            return carry

        lax.fori_loop(0, nch // nb2, body, 0)
        for j in range(NBUF):
            c = (nch // nb2) * nb2 + j

            @pl.when(c < nch)
            def _():
                slot(c, j)
        plsc.subcore_barrier()
        pltpu.sync_copy(acc.at[pl.ds(sid * STRIPE, STRIPE)],
                        out_hbm.at[h, cid, pl.ds(sid * STRIPE, STRIPE)])


def _make_mesh():
    return plsc.VectorSubcoreMesh(core_axis_name="c", subcore_axis_name="s")


def _sc_deg(edge3, ones16, zeros16):
    return pl.kernel(
        _sc_deg_body,
        out_type=jax.ShapeDtypeStruct((NC, NPAD, 16), jnp.float32),
        mesh=_make_mesh(),
        compiler_params=pltpu.CompilerParams(use_tc_tiling_on_sc=False),
        scratch_types=[
            pltpu.VMEM((KB, CH), jnp.int32),
            pltpu.VMEM((CH, 16), jnp.float32),
            pltpu.VMEM_SHARED((NPAD, 16), jnp.float32),
        ],
    )(edge3, ones16, zeros16)


def _sc_agg(gs, edge3, zeros):
    nh = len(gs)
    d = gs[0].shape[1]
    return pl.kernel(
        functools.partial(_sc_agg_body, nh),
        out_type=jax.ShapeDtypeStruct((nh, NC, NPAD, d), jnp.float32),
        mesh=_make_mesh(),
        compiler_params=pltpu.CompilerParams(use_tc_tiling_on_sc=False),
        scratch_types=(
            [pltpu.VMEM((KB, CH), jnp.int32),
             pltpu.VMEM((KB, CH), jnp.int32)]
            + [pltpu.VMEM((CH, d), jnp.float32)] * (2 * NBUF)
            + [pltpu.VMEM_SHARED((NPAD, d), jnp.float32)]
            + [pltpu.SemaphoreType.DMA] * (4 * NBUF)
        ),
    )(*gs, edge3, zeros)


def _rows(d):
    return pl.BlockSpec((BLK, d), lambda i: (i, 0))


def _full(r, c):
    return pl.BlockSpec((r, c), lambda i: (0, 0))


def _degp_spec():
    return pl.BlockSpec((2, BLK, 16), lambda i: (0, i, 0))


def _dis_of(degp_ref):
    deg = degp_ref[0, :, 0:1] + degp_ref[1, :, 0:1] + 1.0
    return lax.rsqrt(jnp.maximum(deg, 1.0))


def _tc_a_body(x_ref, w_ref, degp_ref, gl_ref, gr_ref):
    dis = _dis_of(degp_ref)
    hw = jnp.dot(x_ref[...], w_ref[...], preferred_element_type=jnp.float32)
    g = hw * dis
    gl_ref[...] = g[:, :64]
    gr_ref[...] = g[:, 64:]


def _tc_a(x, w, degp):
    d_in, d_out = w.shape
    return pl.pallas_call(
        _tc_a_body,
        grid=(N // BLK,),
        in_specs=[_rows(d_in), _full(d_in, d_out), _degp_spec()],
        out_specs=[_rows(64), _rows(64)],
        out_shape=[jax.ShapeDtypeStruct((N, 64), jnp.float32),
                   jax.ShapeDtypeStruct((N, 64), jnp.float32)],
    )(x, w, degp)


def _tc_comb_body(a_ref, gl_ref, gr_ref, degp_ref, b_ref, pre_ref, st_ref):
    i = pl.program_id(0)
    dis = _dis_of(degp_ref)
    prel = (a_ref[0, 0] + a_ref[0, 1] + gl_ref[...]) * dis + b_ref[0:1, :64]
    prer = (a_ref[1, 0] + a_ref[1, 1] + gr_ref[...]) * dis + b_ref[0:1, 64:]
    pre = jnp.concatenate([prel, prer], axis=1)
    pre_ref[...] = pre
    @pl.when(i == 0)
    def _():
        st_ref[...] = jnp.zeros_like(st_ref)
    s = jnp.sum(pre, axis=0, keepdims=True)
    sq = jnp.sum(pre * pre, axis=0, keepdims=True)
    pad = jnp.zeros((6, pre.shape[1]), jnp.float32)
    st_ref[...] += jnp.concatenate([s, sq, pad], axis=0)


def _tc_comb(a, gl, gr, degp, b8):
    return pl.pallas_call(
        _tc_comb_body,
        grid=(N // BLK,),
        in_specs=[pl.BlockSpec((2, 2, BLK, 64), lambda i: (0, 0, i, 0)),
                  _rows(64), _rows(64), _degp_spec(), _full(8, 128)],
        out_specs=[_rows(128), _full(8, 128)],
        out_shape=[jax.ShapeDtypeStruct((N, 128), jnp.float32),
                   jax.ShapeDtypeStruct((8, 128), jnp.float32)],
    )(a, gl, gr, degp, b8)


def _tc_bnmm_body(pre_ref, st_ref, gam_ref, bet_ref, w_ref, degp_ref, *out_refs):
    m = st_ref[0:1, :] / N
    v = st_ref[1:2, :] / N - m * m
    h = gam_ref[0:1, :] * (pre_ref[...] - m) * lax.rsqrt(v + BN_EPS) + bet_ref[0:1, :]
    h = jnp.maximum(h, 0.0)
    dis = _dis_of(degp_ref)
    g = jnp.dot(h, w_ref[...], preferred_element_type=jnp.float32) * dis
    if len(out_refs) == 1:
        out_refs[0][...] = g
    else:
        out_refs[0][...] = g[:, :64]
        out_refs[1][...] = g[:, 64:]


def _tc_bnmm(pre, st, gam8, bet8, w, degp):
    d_in, d_out = w.shape
    if d_out == 128:
        out_specs = [_rows(64), _rows(64)]
        out_shape = [jax.ShapeDtypeStruct((N, 64), jnp.float32),
                     jax.ShapeDtypeStruct((N, 64), jnp.float32)]
    else:
        out_specs = _rows(d_out)
        out_shape = jax.ShapeDtypeStruct((N, d_out), jnp.float32)
    return pl.pallas_call(
        _tc_bnmm_body,
        grid=(N // BLK,),
        in_specs=[_rows(d_in), _full(8, d_in), _full(8, d_in), _full(8, d_in),
                  _full(d_in, d_out), _degp_spec()],
        out_specs=out_specs,
        out_shape=out_shape,
    )(pre, st, gam8, bet8, w, degp)


def _tc_final_body(a_ref, g_ref, degp_ref, b_ref, out_ref):
    dis = _dis_of(degp_ref)
    out_ref[...] = (a_ref[0, 0] + a_ref[0, 1] + g_ref[...]) * dis + b_ref[0:1, :]


def _tc_final(a, g, degp, b8):
    d = g.shape[1]
    return pl.pallas_call(
        _tc_final_body,
        grid=(N // BLK,),
        in_specs=[pl.BlockSpec((1, 2, BLK, d), lambda i: (0, 0, i, 0)),
                  _rows(d), _degp_spec(), _full(8, d)],
        out_specs=_rows(d),
        out_shape=jax.ShapeDtypeStruct((N, d), jnp.float32),
    )(a, g, degp, b8)


def _row8(v):
    return jnp.broadcast_to(v.reshape(1, -1), (8, v.shape[0]))


def kernel(x, edge_index, W1, b1, g1, be1, W2, b2, g2, be2, W3, b3):
    edge3 = edge_index.reshape(2, CPT, CH)
    zeros64 = jnp.zeros((STRIPE, 64), jnp.float32)
    zeros16 = jnp.zeros((STRIPE, 16), jnp.float32)
    ones16 = jnp.ones((CH, 16), jnp.float32)

    degp = _sc_deg(edge3, ones16, zeros16)

    gl, gr = _tc_a(x, W1, degp)
    a = _sc_agg((gl, gr), edge3, zeros64)
    pre, st = _tc_comb(a, gl, gr, degp, _row8(b1))
    gl, gr = _tc_bnmm(pre, st, _row8(g1), _row8(be1), W2, degp)
    a = _sc_agg((gl, gr), edge3, zeros64)
    pre, st = _tc_comb(a, gl, gr, degp, _row8(b2))
    gx = _tc_bnmm(pre, st, _row8(g2), _row8(be2), W3, degp)
    a = _sc_agg((gx,), edge3, zeros64)
    return _tc_final(a, gx, degp, _row8(b3))


# speedup vs baseline: 1.0497x; 1.0497x over previous
import functools

import jax
import jax.numpy as jnp
from jax import lax
from jax.experimental import pallas as pl
from jax.experimental.pallas import tpu as pltpu
from jax.experimental.pallas import tpu_sc as plsc

N = 10000
NPAD = 10112
E = 320000
CH = 128
CPT = E // CH
NC, NS = 2, 16
NW = NC * NS
KA, KB = 76, 80
NBUF = 4
STRIPE = NPAD // NS
BN_EPS = 1e-5
BLK = 1000


def _tile_chunks(cid, sid):
    wid = sid * NC + cid
    nch = jnp.where(wid < 15, KA, KB)
    base = jnp.where(wid < 15, wid * KA, 15 * KA + (wid - 15) * KB)
    return nch, base


def _sc_deg_body(edge_hbm, ones_hbm, zeros_hbm, out_hbm, dst_v, ones_v, acc):
    cid = lax.axis_index("c")
    sid = lax.axis_index("s")
    nch, base = _tile_chunks(cid, sid)
    pltpu.sync_copy(edge_hbm.at[1, pl.ds(base, KB)], dst_v)
    pltpu.sync_copy(ones_hbm, ones_v)
    pltpu.sync_copy(zeros_hbm, acc.at[pl.ds(sid * STRIPE, STRIPE)])
    plsc.subcore_barrier()

    def body(c, carry):
        pltpu.sync_copy(ones_v, acc.at[dst_v.at[c]], add=True)
        return carry

    lax.fori_loop(0, nch, body, 0)
    plsc.subcore_barrier()
    pltpu.sync_copy(acc.at[pl.ds(sid * STRIPE, STRIPE)],
                    out_hbm.at[cid, pl.ds(sid * STRIPE, STRIPE)])


def _sc_agg_body(nh, *refs):
    g_hbms = refs[:nh]
    (edge_hbm, zeros_hbm, out_hbm,
     src_v, dst_v, buf0, buf1, buf2, buf3,
     acc, sem0, sem1, sem2, sem3) = refs[nh:]
    cid = lax.axis_index("c")
    sid = lax.axis_index("s")
    nch, base = _tile_chunks(cid, sid)
    bufs = (buf0, buf1, buf2, buf3)
    sems = (sem0, sem1, sem2, sem3)
    pltpu.sync_copy(edge_hbm.at[0, pl.ds(base, KB)], src_v)
    pltpu.sync_copy(edge_hbm.at[1, pl.ds(base, KB)], dst_v)
    for h, g_hbm in enumerate(g_hbms):
        pltpu.sync_copy(zeros_hbm, acc.at[pl.ds(sid * STRIPE, STRIPE)])
        plsc.subcore_barrier()
        for b in range(NBUF):
            pltpu.async_copy(g_hbm.at[src_v.at[b]], bufs[b], sems[b])

        def body(t, carry):
            for b in range(NBUF):
                c = NBUF * t + b
                pltpu.make_async_copy(g_hbm.at[src_v.at[c]],
                                      bufs[b], sems[b]).wait()
                pltpu.sync_copy(bufs[b], acc.at[dst_v.at[c]], add=True)

                @pl.when(c + NBUF < nch)
                def _():
                    pltpu.async_copy(g_hbm.at[src_v.at[c + NBUF]],
                                     bufs[b], sems[b])
            return carry

        lax.fori_loop(0, nch // NBUF, body, 0)
        plsc.subcore_barrier()
        pltpu.sync_copy(acc.at[pl.ds(sid * STRIPE, STRIPE)],
                        out_hbm.at[h, cid, pl.ds(sid * STRIPE, STRIPE)])


def _make_mesh():
    return plsc.VectorSubcoreMesh(core_axis_name="c", subcore_axis_name="s")


def _sc_deg(edge3, ones16, zeros16):
    return pl.kernel(
        _sc_deg_body,
        out_type=jax.ShapeDtypeStruct((NC, NPAD, 16), jnp.float32),
        mesh=_make_mesh(),
        compiler_params=pltpu.CompilerParams(use_tc_tiling_on_sc=False),
        scratch_types=[
            pltpu.VMEM((KB, CH), jnp.int32),
            pltpu.VMEM((CH, 16), jnp.float32),
            pltpu.VMEM_SHARED((NPAD, 16), jnp.float32),
        ],
    )(edge3, ones16, zeros16)


def _sc_agg(gs, edge3, zeros):
    nh = len(gs)
    d = gs[0].shape[1]
    return pl.kernel(
        functools.partial(_sc_agg_body, nh),
        out_type=jax.ShapeDtypeStruct((nh, NC, NPAD, d), jnp.float32),
        mesh=_make_mesh(),
        compiler_params=pltpu.CompilerParams(use_tc_tiling_on_sc=False),
        scratch_types=[
            pltpu.VMEM((KB, CH), jnp.int32),
            pltpu.VMEM((KB, CH), jnp.int32),
            pltpu.VMEM((CH, d), jnp.float32),
            pltpu.VMEM((CH, d), jnp.float32),
            pltpu.VMEM((CH, d), jnp.float32),
            pltpu.VMEM((CH, d), jnp.float32),
            pltpu.VMEM_SHARED((NPAD, d), jnp.float32),
            pltpu.SemaphoreType.DMA,
            pltpu.SemaphoreType.DMA,
            pltpu.SemaphoreType.DMA,
            pltpu.SemaphoreType.DMA,
        ],
    )(*gs, edge3, zeros)


def _rows(d):
    return pl.BlockSpec((BLK, d), lambda i: (i, 0))


def _full(r, c):
    return pl.BlockSpec((r, c), lambda i: (0, 0))


def _degp_spec():
    return pl.BlockSpec((2, BLK, 16), lambda i: (0, i, 0))


def _dis_of(degp_ref):
    deg = degp_ref[0, :, 0:1] + degp_ref[1, :, 0:1] + 1.0
    return lax.rsqrt(jnp.maximum(deg, 1.0))


def _tc_a_body(x_ref, w_ref, degp_ref, gl_ref, gr_ref):
    dis = _dis_of(degp_ref)
    hw = jnp.dot(x_ref[...], w_ref[...], preferred_element_type=jnp.float32)
    g = hw * dis
    gl_ref[...] = g[:, :64]
    gr_ref[...] = g[:, 64:]


def _tc_a(x, w, degp):
    d_in, d_out = w.shape
    return pl.pallas_call(
        _tc_a_body,
        grid=(N // BLK,),
        in_specs=[_rows(d_in), _full(d_in, d_out), _degp_spec()],
        out_specs=[_rows(64), _rows(64)],
        out_shape=[jax.ShapeDtypeStruct((N, 64), jnp.float32),
                   jax.ShapeDtypeStruct((N, 64), jnp.float32)],
    )(x, w, degp)


def _tc_cb_body(a_ref, gl_ref, gr_ref, degp_ref, b_ref, gam_ref, bet_ref,
                w_ref, *rest):
    out_refs, pre_s, st_s = rest[:-2], rest[-2], rest[-1]
    p = pl.program_id(0)
    i = pl.program_id(1)

    @pl.when(p == 0)
    def _():
        dis = _dis_of(degp_ref)
        prel = (a_ref[0, 0] + a_ref[0, 1] + gl_ref[...]) * dis + b_ref[0:1, :64]
        prer = (a_ref[1, 0] + a_ref[1, 1] + gr_ref[...]) * dis + b_ref[0:1, 64:]
        pre = jnp.concatenate([prel, prer], axis=1)
        pre_s[pl.ds(i * BLK, BLK), :] = pre

        @pl.when(i == 0)
        def _():
            st_s[...] = jnp.zeros_like(st_s)
        s = jnp.sum(pre, axis=0, keepdims=True)
        sq = jnp.sum(pre * pre, axis=0, keepdims=True)
        pad = jnp.zeros((6, 128), jnp.float32)
        st_s[...] += jnp.concatenate([s, sq, pad], axis=0)
        for r in out_refs:
            r[...] = jnp.zeros_like(r)

    @pl.when(p == 1)
    def _():
        pre = pre_s[pl.ds(i * BLK, BLK), :]
        m = st_s[0:1, :] / N
        v = st_s[1:2, :] / N - m * m
        h = gam_ref[0:1, :] * (pre - m) * lax.rsqrt(v + BN_EPS) + bet_ref[0:1, :]
        h = jnp.maximum(h, 0.0)
        dis = _dis_of(degp_ref)
        g = jnp.dot(h, w_ref[...], preferred_element_type=jnp.float32) * dis
        if len(out_refs) == 1:
            out_refs[0][...] = g
        else:
            out_refs[0][...] = g[:, :64]
            out_refs[1][...] = g[:, 64:]


def _tc_cb(a, gl, gr, degp, b8, gam8, bet8, w):
    d_in, d_out = w.shape
    ph = lambda p, i: jnp.where(p == 0, i, 0)
    ph1 = lambda p, i: jnp.where(p == 1, i, 0)
    if d_out == 128:
        out_specs = [pl.BlockSpec((BLK, 64), lambda p, i: (ph1(p, i), 0)),
                     pl.BlockSpec((BLK, 64), lambda p, i: (ph1(p, i), 0))]
        out_shape = [jax.ShapeDtypeStruct((N, 64), jnp.float32),
                     jax.ShapeDtypeStruct((N, 64), jnp.float32)]
    else:
        out_specs = pl.BlockSpec((BLK, d_out), lambda p, i: (ph1(p, i), 0))
        out_shape = jax.ShapeDtypeStruct((N, d_out), jnp.float32)
    return pl.pallas_call(
        _tc_cb_body,
        grid=(2, N // BLK),
        in_specs=[pl.BlockSpec((2, 2, BLK, 64), lambda p, i: (0, 0, ph(p, i), 0)),
                  pl.BlockSpec((BLK, 64), lambda p, i: (ph(p, i), 0)),
                  pl.BlockSpec((BLK, 64), lambda p, i: (ph(p, i), 0)),
                  pl.BlockSpec((2, BLK, 16), lambda p, i: (0, i, 0)),
                  pl.BlockSpec((8, 128), lambda p, i: (0, 0)),
                  pl.BlockSpec((8, 128), lambda p, i: (0, 0)),
                  pl.BlockSpec((8, 128), lambda p, i: (0, 0)),
                  pl.BlockSpec((d_in, d_out), lambda p, i: (0, 0))],
        out_specs=out_specs,
        out_shape=out_shape,
        scratch_shapes=[pltpu.VMEM((N, 128), jnp.float32),
                        pltpu.VMEM((8, 128), jnp.float32)],
    )(a, gl, gr, degp, b8, gam8, bet8, w)


def _tc_final_body(a_ref, g_ref, degp_ref, b_ref, out_ref):
    dis = _dis_of(degp_ref)
    out_ref[...] = (a_ref[0, 0] + a_ref[0, 1] + g_ref[...]) * dis + b_ref[0:1, :]


def _tc_final(a, g, degp, b8):
    d = g.shape[1]
    return pl.pallas_call(
        _tc_final_body,
        grid=(N // BLK,),
        in_specs=[pl.BlockSpec((1, 2, BLK, d), lambda i: (0, 0, i, 0)),
                  _rows(d), _degp_spec(), _full(8, d)],
        out_specs=_rows(d),
        out_shape=jax.ShapeDtypeStruct((N, d), jnp.float32),
    )(a, g, degp, b8)


def _row8(v):
    return jnp.broadcast_to(v.reshape(1, -1), (8, v.shape[0]))


def kernel(x, edge_index, W1, b1, g1, be1, W2, b2, g2, be2, W3, b3):
    edge3 = edge_index.reshape(2, CPT, CH)
    zeros64 = jnp.zeros((STRIPE, 64), jnp.float32)
    zeros16 = jnp.zeros((STRIPE, 16), jnp.float32)
    ones16 = jnp.ones((CH, 16), jnp.float32)

    degp = _sc_deg(edge3, ones16, zeros16)

    gl, gr = _tc_a(x, W1, degp)
    a = _sc_agg((gl, gr), edge3, zeros64)
    gl, gr = _tc_cb(a, gl, gr, degp, _row8(b1), _row8(g1), _row8(be1), W2)
    a = _sc_agg((gl, gr), edge3, zeros64)
    gx = _tc_cb(a, gl, gr, degp, _row8(b2), _row8(g2), _row8(be2), W3)
    a = _sc_agg((gx,), edge3, zeros64)
    return _tc_final(a, gx, degp, _row8(b3))


# speedup vs baseline: 1.0784x; 1.0273x over previous
import functools

import jax
import jax.numpy as jnp
from jax import lax
from jax.experimental import pallas as pl
from jax.experimental.pallas import tpu as pltpu
from jax.experimental.pallas import tpu_sc as plsc

N = 10000
NPAD = 10112
E = 320000
CH = 128
CPT = E // CH
NC, NS = 2, 16
NW = NC * NS
KA, KB = 76, 80
NBUF = 4
STRIPE = NPAD // NS
BN_EPS = 1e-5
BLK = 2000


def _tile_chunks(cid, sid):
    wid = sid * NC + cid
    nch = jnp.where(wid < 15, KA, KB)
    base = jnp.where(wid < 15, wid * KA, 15 * KA + (wid - 15) * KB)
    return nch, base


def _sc_deg_body(edge_hbm, ones_hbm, zeros_hbm, out_hbm, dst_v, ones_v, acc):
    cid = lax.axis_index("c")
    sid = lax.axis_index("s")
    nch, base = _tile_chunks(cid, sid)
    pltpu.sync_copy(edge_hbm.at[1, pl.ds(base, KB)], dst_v)
    pltpu.sync_copy(ones_hbm, ones_v)
    pltpu.sync_copy(zeros_hbm, acc.at[pl.ds(sid * STRIPE, STRIPE)])
    plsc.subcore_barrier()

    def body(c, carry):
        pltpu.sync_copy(ones_v, acc.at[dst_v.at[c]], add=True)
        return carry

    lax.fori_loop(0, nch, body, 0)
    plsc.subcore_barrier()
    pltpu.sync_copy(acc.at[pl.ds(sid * STRIPE, STRIPE)],
                    out_hbm.at[cid, pl.ds(sid * STRIPE, STRIPE)])


def _sc_agg_body(nh, *refs):
    g_hbms = refs[:nh]
    (edge_hbm, zeros_hbm, out_hbm,
     src_v, dst_v, buf0, buf1, buf2, buf3,
     acc, sem0, sem1, sem2, sem3) = refs[nh:]
    cid = lax.axis_index("c")
    sid = lax.axis_index("s")
    nch, base = _tile_chunks(cid, sid)
    bufs = (buf0, buf1, buf2, buf3)
    sems = (sem0, sem1, sem2, sem3)
    pltpu.sync_copy(edge_hbm.at[0, pl.ds(base, KB)], src_v)
    pltpu.sync_copy(edge_hbm.at[1, pl.ds(base, KB)], dst_v)
    for h, g_hbm in enumerate(g_hbms):
        pltpu.sync_copy(zeros_hbm, acc.at[pl.ds(sid * STRIPE, STRIPE)])
        plsc.subcore_barrier()
        for b in range(NBUF):
            pltpu.async_copy(g_hbm.at[src_v.at[b]], bufs[b], sems[b])

        def body(t, carry):
            for b in range(NBUF):
                c = NBUF * t + b
                pltpu.make_async_copy(g_hbm.at[src_v.at[c]],
                                      bufs[b], sems[b]).wait()
                pltpu.sync_copy(bufs[b], acc.at[dst_v.at[c]], add=True)

                @pl.when(c + NBUF < nch)
                def _():
                    pltpu.async_copy(g_hbm.at[src_v.at[c + NBUF]],
                                     bufs[b], sems[b])
            return carry

        lax.fori_loop(0, nch // NBUF, body, 0)
        plsc.subcore_barrier()
        pltpu.sync_copy(acc.at[pl.ds(sid * STRIPE, STRIPE)],
                        out_hbm.at[h, cid, pl.ds(sid * STRIPE, STRIPE)])


def _make_mesh():
    return plsc.VectorSubcoreMesh(core_axis_name="c", subcore_axis_name="s")


def _sc_deg(edge3, ones16, zeros16):
    return pl.kernel(
        _sc_deg_body,
        out_type=jax.ShapeDtypeStruct((NC, NPAD, 16), jnp.float32),
        mesh=_make_mesh(),
        compiler_params=pltpu.CompilerParams(use_tc_tiling_on_sc=False),
        scratch_types=[
            pltpu.VMEM((KB, CH), jnp.int32),
            pltpu.VMEM((CH, 16), jnp.float32),
            pltpu.VMEM_SHARED((NPAD, 16), jnp.float32),
        ],
    )(edge3, ones16, zeros16)


def _sc_agg(gs, edge3, zeros):
    nh = len(gs)
    d = gs[0].shape[1]
    return pl.kernel(
        functools.partial(_sc_agg_body, nh),
        out_type=jax.ShapeDtypeStruct((nh, NC, NPAD, d), jnp.float32),
        mesh=_make_mesh(),
        compiler_params=pltpu.CompilerParams(use_tc_tiling_on_sc=False),
        scratch_types=[
            pltpu.VMEM((KB, CH), jnp.int32),
            pltpu.VMEM((KB, CH), jnp.int32),
            pltpu.VMEM((CH, d), jnp.float32),
            pltpu.VMEM((CH, d), jnp.float32),
            pltpu.VMEM((CH, d), jnp.float32),
            pltpu.VMEM((CH, d), jnp.float32),
            pltpu.VMEM_SHARED((NPAD, d), jnp.float32),
            pltpu.SemaphoreType.DMA,
            pltpu.SemaphoreType.DMA,
            pltpu.SemaphoreType.DMA,
            pltpu.SemaphoreType.DMA,
        ],
    )(*gs, edge3, zeros)


def _rows(d):
    return pl.BlockSpec((BLK, d), lambda i: (i, 0))


def _full(r, c):
    return pl.BlockSpec((r, c), lambda i: (0, 0))


def _degp_spec():
    return pl.BlockSpec((2, BLK, 16), lambda i: (0, i, 0))


def _dis_of(degp_ref):
    deg = degp_ref[0, :, 0:1] + degp_ref[1, :, 0:1] + 1.0
    return lax.rsqrt(jnp.maximum(deg, 1.0))


def _tc_a_body(x_ref, w_ref, degp_ref, gl_ref, gr_ref):
    dis = _dis_of(degp_ref)
    hw = jnp.dot(x_ref[...], w_ref[...], preferred_element_type=jnp.float32)
    g = hw * dis
    gl_ref[...] = g[:, :64]
    gr_ref[...] = g[:, 64:]


def _tc_a(x, w, degp):
    d_in, d_out = w.shape
    return pl.pallas_call(
        _tc_a_body,
        grid=(N // BLK,),
        in_specs=[_rows(d_in), _full(d_in, d_out), _degp_spec()],
        out_specs=[_rows(64), _rows(64)],
        out_shape=[jax.ShapeDtypeStruct((N, 64), jnp.float32),
                   jax.ShapeDtypeStruct((N, 64), jnp.float32)],
    )(x, w, degp)


def _tc_cb_body(a_ref, gl_ref, gr_ref, degp_ref, b_ref, gam_ref, bet_ref,
                w_ref, *rest):
    out_refs, pre_s, st_s = rest[:-2], rest[-2], rest[-1]
    p = pl.program_id(0)
    i = pl.program_id(1)

    @pl.when(p == 0)
    def _():
        dis = _dis_of(degp_ref)
        prel = (a_ref[0, 0] + a_ref[0, 1] + gl_ref[...]) * dis + b_ref[0:1, :64]
        prer = (a_ref[1, 0] + a_ref[1, 1] + gr_ref[...]) * dis + b_ref[0:1, 64:]
        pre = jnp.concatenate([prel, prer], axis=1)
        pre_s[pl.ds(i * BLK, BLK), :] = pre

        @pl.when(i == 0)
        def _():
            st_s[...] = jnp.zeros_like(st_s)
        s = jnp.sum(pre, axis=0, keepdims=True)
        sq = jnp.sum(pre * pre, axis=0, keepdims=True)
        pad = jnp.zeros((6, 128), jnp.float32)
        st_s[...] += jnp.concatenate([s, sq, pad], axis=0)
        for r in out_refs:
            r[...] = jnp.zeros_like(r)

    @pl.when(p == 1)
    def _():
        pre = pre_s[pl.ds(i * BLK, BLK), :]
        m = st_s[0:1, :] / N
        v = st_s[1:2, :] / N - m * m
        h = gam_ref[0:1, :] * (pre - m) * lax.rsqrt(v + BN_EPS) + bet_ref[0:1, :]
        h = jnp.maximum(h, 0.0)
        dis = _dis_of(degp_ref)
        g = jnp.dot(h, w_ref[...], preferred_element_type=jnp.float32) * dis
        if len(out_refs) == 1:
            out_refs[0][...] = g
        else:
            out_refs[0][...] = g[:, :64]
            out_refs[1][...] = g[:, 64:]


def _tc_cb(a, gl, gr, degp, b8, gam8, bet8, w):
    d_in, d_out = w.shape
    ph = lambda p, i: jnp.where(p == 0, i, 0)
    ph1 = lambda p, i: jnp.where(p == 1, i, 0)
    if d_out == 128:
        out_specs = [pl.BlockSpec((BLK, 64), lambda p, i: (ph1(p, i), 0)),
                     pl.BlockSpec((BLK, 64), lambda p, i: (ph1(p, i), 0))]
        out_shape = [jax.ShapeDtypeStruct((N, 64), jnp.float32),
                     jax.ShapeDtypeStruct((N, 64), jnp.float32)]
    else:
        out_specs = pl.BlockSpec((BLK, d_out), lambda p, i: (ph1(p, i), 0))
        out_shape = jax.ShapeDtypeStruct((N, d_out), jnp.float32)
    return pl.pallas_call(
        _tc_cb_body,
        grid=(2, N // BLK),
        in_specs=[pl.BlockSpec((2, 2, BLK, 64), lambda p, i: (0, 0, ph(p, i), 0)),
                  pl.BlockSpec((BLK, 64), lambda p, i: (ph(p, i), 0)),
                  pl.BlockSpec((BLK, 64), lambda p, i: (ph(p, i), 0)),
                  pl.BlockSpec((2, BLK, 16), lambda p, i: (0, i, 0)),
                  pl.BlockSpec((8, 128), lambda p, i: (0, 0)),
                  pl.BlockSpec((8, 128), lambda p, i: (0, 0)),
                  pl.BlockSpec((8, 128), lambda p, i: (0, 0)),
                  pl.BlockSpec((d_in, d_out), lambda p, i: (0, 0))],
        out_specs=out_specs,
        out_shape=out_shape,
        scratch_shapes=[pltpu.VMEM((N, 128), jnp.float32),
                        pltpu.VMEM((8, 128), jnp.float32)],
    )(a, gl, gr, degp, b8, gam8, bet8, w)


def _tc_final_body(a_ref, g_ref, degp_ref, b_ref, out_ref):
    dis = _dis_of(degp_ref)
    out_ref[...] = (a_ref[0, 0] + a_ref[0, 1] + g_ref[...]) * dis + b_ref[0:1, :]


def _tc_final(a, g, degp, b8):
    d = g.shape[1]
    return pl.pallas_call(
        _tc_final_body,
        grid=(N // BLK,),
        in_specs=[pl.BlockSpec((1, 2, BLK, d), lambda i: (0, 0, i, 0)),
                  _rows(d), _degp_spec(), _full(8, d)],
        out_specs=_rows(d),
        out_shape=jax.ShapeDtypeStruct((N, d), jnp.float32),
    )(a, g, degp, b8)


def _row8(v):
    return jnp.broadcast_to(v.reshape(1, -1), (8, v.shape[0]))


def kernel(x, edge_index, W1, b1, g1, be1, W2, b2, g2, be2, W3, b3):
    edge3 = edge_index.reshape(2, CPT, CH)
    zeros64 = jnp.zeros((STRIPE, 64), jnp.float32)
    zeros16 = jnp.zeros((STRIPE, 16), jnp.float32)
    ones16 = jnp.ones((CH, 16), jnp.float32)

    degp = _sc_deg(edge3, ones16, zeros16)

    gl, gr = _tc_a(x, W1, degp)
    a = _sc_agg((gl, gr), edge3, zeros64)
    gl, gr = _tc_cb(a, gl, gr, degp, _row8(b1), _row8(g1), _row8(be1), W2)
    a = _sc_agg((gl, gr), edge3, zeros64)
    gx = _tc_cb(a, gl, gr, degp, _row8(b2), _row8(g2), _row8(be2), W3)
    a = _sc_agg((gx,), edge3, zeros64)
    return _tc_final(a, gx, degp, _row8(b3))
